# baseline (speedup 1.0000x reference)
import functools

import jax
import jax.numpy as jnp
from jax import lax
from jax.experimental import pallas as pl
from jax.experimental.pallas import tpu as pltpu
from jax.experimental.pallas import tpu_sc as plsc

_NUM_ROWS = 1_000_000
_DIM = 64
_BATCH = 16384

_NC = 2
_NS = 16
_NW = _NC * _NS
_BPW = _BATCH // _NW


def _gather_body(table_hbm, idx_hbm, out_hbm, idx_v, sem):
    wid = lax.axis_index("s") * _NC + lax.axis_index("c")
    base = wid * _BPW
    pltpu.sync_copy(idx_hbm.at[wid], idx_v.at[pl.ds(0, _BPW)])

    def issue(j, carry):
        row = idx_v[pl.ds(j, 16)][0]
        pltpu.async_copy(
            table_hbm.at[pl.ds(row, 1)],
            out_hbm.at[pl.ds(base + j, 1)],
            sem,
        )
        return carry

    lax.fori_loop(0, _BPW, issue, 0)
    pltpu.make_async_copy(
        table_hbm.at[pl.ds(0, _BPW)],
        out_hbm.at[pl.ds(base, _BPW)],
        sem,
    ).wait()


_sc_gather = pl.kernel(
    _gather_body,
    out_type=jax.ShapeDtypeStruct((_BATCH, _DIM), jnp.float32),
    mesh=plsc.VectorSubcoreMesh(core_axis_name="c", subcore_axis_name="s"),
    scratch_types=[
        pltpu.VMEM((_BPW + 16,), jnp.int32),
        pltpu.SemaphoreType.DMA,
    ],
    compiler_params=pltpu.CompilerParams(use_tc_tiling_on_sc=True),
)


@jax.jit
def kernel(batchInds, physiologicalProfile):
    idx2 = batchInds.reshape(_NW, _BPW)
    return _sc_gather(physiologicalProfile, idx2)


# speedup vs baseline: 1.2659x; 1.2659x over previous
import functools

import jax
import jax.numpy as jnp
from jax import lax
from jax.experimental import pallas as pl
from jax.experimental.pallas import tpu as pltpu
from jax.experimental.pallas import tpu_sc as plsc

_NUM_ROWS = 1_000_000
_DIM = 64
_BATCH = 16384

_NC = 2
_NS = 16
_NW = _NC * _NS
_BPW = _BATCH // _NW
_GRP = _NUM_ROWS // 8


def _gather_body(table3_hbm, idx_hbm, out_hbm, idx_v, sem_a, sem_b):
    wid = lax.axis_index("s") * _NC + lax.axis_index("c")
    base = wid * _BPW
    pltpu.sync_copy(idx_hbm.at[wid], idx_v.at[pl.ds(0, _BPW)])

    def issue(j, carry):
        row = idx_v[pl.ds(2 * j, 16)][0]
        g = lax.shift_right_logical(row, 3)
        s = lax.bitwise_and(row, 7)
        pltpu.async_copy(
            table3_hbm.at[g, pl.ds(s, 1)],
            out_hbm.at[pl.ds(base + 2 * j, 1)],
            sem_a,
        )
        row2 = idx_v[pl.ds(2 * j + 1, 16)][0]
        g2 = lax.shift_right_logical(row2, 3)
        s2 = lax.bitwise_and(row2, 7)
        pltpu.async_copy(
            table3_hbm.at[g2, pl.ds(s2, 1)],
            out_hbm.at[pl.ds(base + 2 * j + 1, 1)],
            sem_b,
        )
        return carry

    lax.fori_loop(0, _BPW // 2, issue, 0)
    pltpu.make_async_copy(
        out_hbm.at[pl.ds(0, _BPW // 2)],
        out_hbm.at[pl.ds(base, _BPW // 2)],
        sem_a,
    ).wait()
    pltpu.make_async_copy(
        out_hbm.at[pl.ds(0, _BPW // 2)],
        out_hbm.at[pl.ds(base + _BPW // 2, _BPW // 2)],
        sem_b,
    ).wait()


_sc_gather = pl.kernel(
    _gather_body,
    out_type=jax.ShapeDtypeStruct((_BATCH, _DIM), jnp.float32),
    mesh=plsc.VectorSubcoreMesh(core_axis_name="c", subcore_axis_name="s"),
    scratch_types=[
        pltpu.VMEM((_BPW + 16,), jnp.int32),
        pltpu.SemaphoreType.DMA,
        pltpu.SemaphoreType.DMA,
    ],
    compiler_params=pltpu.CompilerParams(use_tc_tiling_on_sc=True),
)


@jax.jit
def kernel(batchInds, physiologicalProfile):
    table3 = physiologicalProfile.reshape(_GRP, 8, _DIM)
    idx2 = batchInds.reshape(_NW, _BPW)
    return _sc_gather(table3, idx2)
